# Initial kernel scaffold; baseline (speedup 1.0000x reference)
import jax
import jax.numpy as jnp
from jax.experimental import pallas as pl


def kernel(x, Wg, bg, W1, b1, W2, b2, Ws1, bs1, Ws2, bs2):
    raise NotImplementedError("write your pallas kernel here")


# speedup vs baseline: 2.0421x; 2.0421x over previous
import functools

import jax
import jax.numpy as jnp
from jax import lax
from jax.experimental import pallas as pl
from jax.experimental.pallas import tpu as pltpu
from jax.experimental.pallas import tpu_sc as plsc

D_MODEL = 768
D_FF = 3072
E = 8
K = 2
T = 2048
P = T * K
C = P // E
DUMMY = E * C
ROWS = E * C + 512
LANES = 128
RB = 256
NRB = T // RB

NEG = -1e30

NC, NS = 2, 16
NW = NC * NS
TPW = T // NW

@functools.cache
def _sc_mesh():
    return plsc.VectorSubcoreMesh(
        core_axis_name="c", subcore_axis_name="s",
        num_cores=NC, num_subcores=NS)


def _router_body(xf_ref, wg_ref, bg_ref, mi_ref, kwx_ref, a_ref, s_ref, f_ref):
    f32 = jnp.float32
    xf = xf_ref[...]
    logits = jnp.dot(xf, wg_ref[...], preferred_element_type=f32) + bg_ref[...]
    lane = lax.broadcasted_iota(jnp.int32, (T, LANES), 1)

    m1 = jnp.max(logits, axis=1, keepdims=True)
    i1 = jnp.min(jnp.where(logits == m1, lane, LANES - 1), axis=1,
                 keepdims=True)
    l2m = jnp.where(lane == i1, NEG, logits)
    m2 = jnp.max(l2m, axis=1, keepdims=True)
    i2 = jnp.min(jnp.where(l2m == m2, lane, LANES - 1), axis=1, keepdims=True)

    r = jnp.exp(m2 - m1)
    w1 = 1.0 / (1.0 + r)
    w2 = r * w1

    a_ref[...] = ((lane == i1) | (lane == i2)).astype(f32)

    rloc = lax.broadcasted_iota(jnp.int32, (RB, RB), 0)
    cloc = lax.broadcasted_iota(jnp.int32, (RB, RB), 1)
    stril = (cloc < rloc).astype(f32)
    eye = (cloc == rloc).astype(f32)

    def cum_step(b, carry):
        off = pl.multiple_of(b * RB, RB)
        ablk = a_ref[pl.ds(off, RB), :]
        s_ref[pl.ds(off, RB), :] = (
            jnp.dot(stril, ablk, preferred_element_type=f32,
                    precision=lax.Precision.HIGHEST) + carry)
        return carry + jnp.sum(ablk, axis=0, keepdims=True)

    lax.fori_loop(0, NRB, cum_step, jnp.zeros((1, LANES), f32))

    s = s_ref[...]
    pos0 = jnp.sum(jnp.where(lane == i1, s, 0.0), axis=1, keepdims=True)
    pos1 = jnp.sum(jnp.where(lane == i2, s, 0.0), axis=1, keepdims=True)

    i1f = i1.astype(f32)
    i2f = i2.astype(f32)
    keep0 = pos0 <= float(C - 1)
    keep1 = pos1 <= float(C - 1)
    lin0 = i1f * float(C) + jnp.minimum(pos0, float(C - 1))
    lin1 = i2f * float(C) + jnp.minimum(pos1, float(C - 1))
    dst0 = jnp.where(keep0, lin0, float(DUMMY))
    dst1 = jnp.where(keep1, lin1, float(DUMMY))
    kw0 = jnp.where(keep0, w1, 0.0)
    kw1 = jnp.where(keep1, w2, 0.0)

    lane32 = lax.broadcasted_iota(jnp.int32, (T, 32), 1)
    kwx_ref[...] = jnp.where(lane32 < 16, kw0, kw1)

    fm = jnp.where(lane == 0, dst0,
         jnp.where(lane == 1, dst1,
         jnp.where(lane == 2, lin0,
         jnp.where(lane == 3, lin1, 0.0))))
    f_ref[...] = fm

    def tr_step(b, carry):
        off = pl.multiple_of(b * RB, RB)
        blk = f_ref[pl.ds(off, RB), :]
        tb = lax.dot_general(blk, eye, (((0,), (0,)), ((), ())),
                             preferred_element_type=f32,
                             precision=lax.Precision.HIGHEST)
        mi_ref[:, pl.ds(off, RB)] = tb[0:8, :].astype(jnp.int32)
        return carry

    lax.fori_loop(0, NRB, tr_step, 0)


def _router(xf, wgp, bgp):
    return pl.pallas_call(
        _router_body,
        out_shape=[
            jax.ShapeDtypeStruct((8, T), jnp.int32),
            jax.ShapeDtypeStruct((T, 32), jnp.float32),
        ],
        scratch_shapes=[
            pltpu.VMEM((T, LANES), jnp.float32),
            pltpu.VMEM((T, LANES), jnp.float32),
            pltpu.VMEM((T, LANES), jnp.float32),
        ],
    )(xf, wgp, bgp)


@functools.cache
def _dispatch_kernel():
    @functools.partial(
        pl.kernel,
        out_type=jax.ShapeDtypeStruct((ROWS, D_MODEL), jnp.float32),
        mesh=_sc_mesh(),
        scratch_types=[
            pltpu.VMEM((TPW, D_MODEL), jnp.float32),
            pltpu.VMEM((TPW,), jnp.int32),
            pltpu.VMEM((TPW,), jnp.int32),
            pltpu.SemaphoreType.DMA,
        ],
    )
    def _dispatch(xf_hbm, mi_hbm, buf_hbm, tok_v, d0_v, d1_v, sem):
        wid = lax.axis_index("s") * NC + lax.axis_index("c")
        base = wid * TPW
        pltpu.sync_copy(xf_hbm.at[pl.ds(base, TPW)], tok_v)
        pltpu.sync_copy(mi_hbm.at[0, pl.ds(base, TPW)], d0_v)
        pltpu.sync_copy(mi_hbm.at[1, pl.ds(base, TPW)], d1_v)
        pltpu.async_copy(tok_v, buf_hbm.at[d0_v], sem).wait()
        pltpu.async_copy(tok_v, buf_hbm.at[d1_v], sem).wait()

    return _dispatch


NFB = 6
FB = D_FF // NFB


def _expert_body(x_ref, w1_ref, b1_ref, w2_ref, b2_ref, out_ref, acc_ref):
    j = pl.program_id(1)
    f32 = jnp.float32
    h = jax.nn.gelu(
        jnp.dot(x_ref[...], w1_ref[0], preferred_element_type=f32)
        + b1_ref[0])
    contrib = jnp.dot(h, w2_ref[0], preferred_element_type=f32)

    @pl.when(j == 0)
    def _():
        acc_ref[...] = contrib + b2_ref[0]

    @pl.when(j > 0)
    def _():
        acc_ref[...] += contrib

    @pl.when(j == NFB - 1)
    def _():
        out_ref[...] = acc_ref[...]


def _experts(buf, W1, b1, W2, b2):
    return pl.pallas_call(
        _expert_body,
        grid=(E, NFB),
        in_specs=[
            pl.BlockSpec((C, D_MODEL), lambda e, j: (e, 0)),
            pl.BlockSpec((1, D_MODEL, FB), lambda e, j: (e, 0, j)),
            pl.BlockSpec((1, 1, FB), lambda e, j: (e, 0, j)),
            pl.BlockSpec((1, FB, D_MODEL), lambda e, j: (e, j, 0)),
            pl.BlockSpec((1, 1, D_MODEL), lambda e, j: (e, 0, 0)),
        ],
        out_specs=pl.BlockSpec((C, D_MODEL), lambda e, j: (e, 0)),
        out_shape=jax.ShapeDtypeStruct((ROWS, D_MODEL), jnp.float32),
        scratch_shapes=[pltpu.VMEM((C, D_MODEL), jnp.float32)],
        compiler_params=pltpu.CompilerParams(
            dimension_semantics=("parallel", "arbitrary")),
    )(buf, W1, b1, W2, b2)


@functools.cache
def _combine_kernel():
    @functools.partial(
        pl.kernel,
        out_type=jax.ShapeDtypeStruct((T, D_MODEL), jnp.float32),
        mesh=_sc_mesh(),
        scratch_types=[
            pltpu.VMEM((TPW,), jnp.int32),
            pltpu.VMEM((TPW,), jnp.int32),
            pltpu.VMEM((TPW, 32), jnp.float32),
            pltpu.VMEM((TPW, D_MODEL), jnp.float32),
            pltpu.VMEM((TPW, D_MODEL), jnp.float32),
            pltpu.SemaphoreType.DMA,
        ],
    )
    def _combine(yb_hbm, mi_hbm, kwx_hbm, out_hbm,
                 l0_v, l1_v, kwx_v, r_v, acc_v, sem):
        wid = lax.axis_index("s") * NC + lax.axis_index("c")
        base = wid * TPW
        pltpu.sync_copy(mi_hbm.at[2, pl.ds(base, TPW)], l0_v)
        pltpu.sync_copy(mi_hbm.at[3, pl.ds(base, TPW)], l1_v)
        pltpu.sync_copy(kwx_hbm.at[pl.ds(base, TPW)], kwx_v)

        pltpu.async_copy(yb_hbm.at[l0_v], r_v, sem).wait()

        def scale0(i, carry):
            w = kwx_v[i, pl.ds(0, 16)]
            for cc in range(D_MODEL // 16):
                sl = pl.ds(cc * 16, 16)
                acc_v[i, sl] = w * r_v[i, sl]
            return carry

        lax.fori_loop(0, TPW, scale0, 0)

        pltpu.async_copy(yb_hbm.at[l1_v], r_v, sem).wait()

        def scale1(i, carry):
            w = kwx_v[i, pl.ds(16, 16)]
            for cc in range(D_MODEL // 16):
                sl = pl.ds(cc * 16, 16)
                acc_v[i, sl] = acc_v[i, sl] + w * r_v[i, sl]
            return carry

        lax.fori_loop(0, TPW, scale1, 0)

        pltpu.sync_copy(acc_v, out_hbm.at[pl.ds(base, TPW)])

    return _combine


def _shared_body(x_ref, w1_ref, b1_ref, w2_ref, b2_ref, sp_ref, out_ref,
                 acc_ref):
    j = pl.program_id(0)
    f32 = jnp.float32
    h = jax.nn.gelu(
        jnp.dot(x_ref[...], w1_ref[...], preferred_element_type=f32)
        + b1_ref[...])
    contrib = jnp.dot(h, w2_ref[...], preferred_element_type=f32)

    @pl.when(j == 0)
    def _():
        acc_ref[...] = contrib + b2_ref[...] + sp_ref[...]

    @pl.when(j > 0)
    def _():
        acc_ref[...] += contrib

    @pl.when(j == NFB - 1)
    def _():
        out_ref[...] = acc_ref[...]


def _shared(xf, Ws1, bs1, Ws2, bs2, sparse):
    return pl.pallas_call(
        _shared_body,
        grid=(NFB,),
        in_specs=[
            pl.BlockSpec((T, D_MODEL), lambda j: (0, 0)),
            pl.BlockSpec((D_MODEL, FB), lambda j: (0, j)),
            pl.BlockSpec((1, FB), lambda j: (0, j)),
            pl.BlockSpec((FB, D_MODEL), lambda j: (j, 0)),
            pl.BlockSpec((1, D_MODEL), lambda j: (0, 0)),
            pl.BlockSpec((T, D_MODEL), lambda j: (0, 0)),
        ],
        out_specs=pl.BlockSpec((T, D_MODEL), lambda j: (0, 0)),
        out_shape=jax.ShapeDtypeStruct((T, D_MODEL), jnp.float32),
        scratch_shapes=[pltpu.VMEM((T, D_MODEL), jnp.float32)],
        compiler_params=pltpu.CompilerParams(
            dimension_semantics=("arbitrary",)),
    )(xf, Ws1, bs1, Ws2, bs2, sparse)


def kernel(x, Wg, bg, W1, b1, W2, b2, Ws1, bs1, Ws2, bs2):
    B, S, d = x.shape
    xf = x.reshape(T, d)

    wgp = jnp.zeros((D_MODEL, LANES), jnp.float32).at[:, :E].set(Wg)
    bgp = jnp.full((1, LANES), NEG, jnp.float32).at[0, :E].set(bg)

    mi, kwx = _router(xf, wgp, bgp)
    buf = _dispatch_kernel()(xf, mi)
    yb = _experts(buf, W1, b1.reshape(E, 1, D_FF), W2,
                  b2.reshape(E, 1, D_MODEL))
    sparse = _combine_kernel()(yb, mi, kwx)
    out = _shared(xf, Ws1, bs1.reshape(1, D_FF), Ws2, bs2.reshape(1, D_MODEL),
                  sparse)
    return out.reshape(B, S, d)


# speedup vs baseline: 2.0491x; 1.0035x over previous
import functools

import jax
import jax.numpy as jnp
from jax import lax
from jax.experimental import pallas as pl
from jax.experimental.pallas import tpu as pltpu
from jax.experimental.pallas import tpu_sc as plsc

D_MODEL = 768
D_FF = 3072
E = 8
K = 2
T = 2048
P = T * K
C = P // E
DUMMY = E * C
ROWS = E * C + 512
LANES = 128
RB = 256
NRB = T // RB

NEG = -1e30

NC, NS = 2, 16
NW = NC * NS
TPW = T // NW

@functools.cache
def _sc_mesh():
    return plsc.VectorSubcoreMesh(
        core_axis_name="c", subcore_axis_name="s",
        num_cores=NC, num_subcores=NS)


def _router_body(xf_ref, wg_ref, bg_ref, mi_ref, kwx_ref, a_ref, s_ref, f_ref):
    f32 = jnp.float32
    xf = xf_ref[...]
    logits = jnp.dot(xf, wg_ref[...], preferred_element_type=f32) + bg_ref[...]
    lane = lax.broadcasted_iota(jnp.int32, (T, LANES), 1)

    m1 = jnp.max(logits, axis=1, keepdims=True)
    i1 = jnp.min(jnp.where(logits == m1, lane, LANES - 1), axis=1,
                 keepdims=True)
    l2m = jnp.where(lane == i1, NEG, logits)
    m2 = jnp.max(l2m, axis=1, keepdims=True)
    i2 = jnp.min(jnp.where(l2m == m2, lane, LANES - 1), axis=1, keepdims=True)

    r = jnp.exp(m2 - m1)
    w1 = 1.0 / (1.0 + r)
    w2 = r * w1

    a_ref[...] = ((lane == i1) | (lane == i2)).astype(f32)

    rloc = lax.broadcasted_iota(jnp.int32, (RB, RB), 0)
    cloc = lax.broadcasted_iota(jnp.int32, (RB, RB), 1)
    stril = (cloc < rloc).astype(f32)
    eye = (cloc == rloc).astype(f32)

    def cum_step(b, carry):
        off = pl.multiple_of(b * RB, RB)
        ablk = a_ref[pl.ds(off, RB), :]
        s_ref[pl.ds(off, RB), :] = (
            jnp.dot(stril, ablk, preferred_element_type=f32,
                    precision=lax.Precision.HIGHEST) + carry)
        return carry + jnp.sum(ablk, axis=0, keepdims=True)

    lax.fori_loop(0, NRB, cum_step, jnp.zeros((1, LANES), f32))

    s = s_ref[...]
    pos0 = jnp.sum(jnp.where(lane == i1, s, 0.0), axis=1, keepdims=True)
    pos1 = jnp.sum(jnp.where(lane == i2, s, 0.0), axis=1, keepdims=True)

    i1f = i1.astype(f32)
    i2f = i2.astype(f32)
    keep0 = pos0 <= float(C - 1)
    keep1 = pos1 <= float(C - 1)
    lin0 = i1f * float(C) + jnp.minimum(pos0, float(C - 1))
    lin1 = i2f * float(C) + jnp.minimum(pos1, float(C - 1))
    dst0 = jnp.where(keep0, lin0, float(DUMMY))
    dst1 = jnp.where(keep1, lin1, float(DUMMY))
    kw0 = jnp.where(keep0, w1, 0.0)
    kw1 = jnp.where(keep1, w2, 0.0)

    lane32 = lax.broadcasted_iota(jnp.int32, (T, 32), 1)
    kwx_ref[...] = jnp.where(lane32 < 16, kw0, kw1)

    fm = jnp.where(lane == 0, dst0,
         jnp.where(lane == 1, dst1,
         jnp.where(lane == 2, lin0,
         jnp.where(lane == 3, lin1, 0.0))))
    f_ref[...] = fm

    def tr_step(b, carry):
        off = pl.multiple_of(b * RB, RB)
        blk = f_ref[pl.ds(off, RB), :]
        tb = lax.dot_general(blk, eye, (((0,), (0,)), ((), ())),
                             preferred_element_type=f32,
                             precision=lax.Precision.HIGHEST)
        mi_ref[:, pl.ds(off, RB)] = tb[0:8, :].astype(jnp.int32)
        return carry

    lax.fori_loop(0, NRB, tr_step, 0)


def _router(xf, wgp, bgp):
    return pl.pallas_call(
        _router_body,
        out_shape=[
            jax.ShapeDtypeStruct((8, T), jnp.int32),
            jax.ShapeDtypeStruct((T, 32), jnp.float32),
        ],
        scratch_shapes=[
            pltpu.VMEM((T, LANES), jnp.float32),
            pltpu.VMEM((T, LANES), jnp.float32),
            pltpu.VMEM((T, LANES), jnp.float32),
        ],
    )(xf, wgp, bgp)


@functools.cache
def _dispatch_kernel():
    @functools.partial(
        pl.kernel,
        out_type=jax.ShapeDtypeStruct((ROWS, D_MODEL), jnp.float32),
        mesh=_sc_mesh(),
        scratch_types=[
            pltpu.VMEM((TPW, D_MODEL), jnp.float32),
            pltpu.VMEM((TPW,), jnp.int32),
            pltpu.VMEM((TPW,), jnp.int32),
            pltpu.SemaphoreType.DMA,
        ],
    )
    def _dispatch(xf_hbm, mi_hbm, buf_hbm, tok_v, d0_v, d1_v, sem):
        wid = lax.axis_index("s") * NC + lax.axis_index("c")
        base = wid * TPW
        pltpu.sync_copy(xf_hbm.at[pl.ds(base, TPW)], tok_v)
        pltpu.sync_copy(mi_hbm.at[0, pl.ds(base, TPW)], d0_v)
        pltpu.sync_copy(mi_hbm.at[1, pl.ds(base, TPW)], d1_v)
        pltpu.async_copy(tok_v, buf_hbm.at[d0_v], sem).wait()
        pltpu.async_copy(tok_v, buf_hbm.at[d1_v], sem).wait()

    return _dispatch


NFB = 6
FB = D_FF // NFB


def _expert_body(x_ref, w1_ref, b1_ref, w2_ref, b2_ref, out_ref, acc_ref):
    j = pl.program_id(1)
    f32 = jnp.float32
    bf16 = jnp.bfloat16
    h = jax.nn.gelu(
        jnp.dot(x_ref[...].astype(bf16), w1_ref[0].astype(bf16),
                preferred_element_type=f32)
        + b1_ref[0])
    contrib = jnp.dot(h.astype(bf16), w2_ref[0].astype(bf16),
                      preferred_element_type=f32)

    @pl.when(j == 0)
    def _():
        acc_ref[...] = contrib + b2_ref[0]

    @pl.when(j > 0)
    def _():
        acc_ref[...] += contrib

    @pl.when(j == NFB - 1)
    def _():
        out_ref[...] = acc_ref[...]


def _experts(buf, W1, b1, W2, b2):
    return pl.pallas_call(
        _expert_body,
        grid=(E, NFB),
        in_specs=[
            pl.BlockSpec((C, D_MODEL), lambda e, j: (e, 0)),
            pl.BlockSpec((1, D_MODEL, FB), lambda e, j: (e, 0, j)),
            pl.BlockSpec((1, 1, FB), lambda e, j: (e, 0, j)),
            pl.BlockSpec((1, FB, D_MODEL), lambda e, j: (e, j, 0)),
            pl.BlockSpec((1, 1, D_MODEL), lambda e, j: (e, 0, 0)),
        ],
        out_specs=pl.BlockSpec((C, D_MODEL), lambda e, j: (e, 0)),
        out_shape=jax.ShapeDtypeStruct((ROWS, D_MODEL), jnp.float32),
        scratch_shapes=[pltpu.VMEM((C, D_MODEL), jnp.float32)],
        compiler_params=pltpu.CompilerParams(
            dimension_semantics=("parallel", "arbitrary")),
    )(buf, W1, b1, W2, b2)


@functools.cache
def _combine_kernel():
    @functools.partial(
        pl.kernel,
        out_type=jax.ShapeDtypeStruct((T, D_MODEL), jnp.float32),
        mesh=_sc_mesh(),
        scratch_types=[
            pltpu.VMEM((TPW,), jnp.int32),
            pltpu.VMEM((TPW,), jnp.int32),
            pltpu.VMEM((TPW, 32), jnp.float32),
            pltpu.VMEM((TPW, D_MODEL), jnp.float32),
            pltpu.VMEM((TPW, D_MODEL), jnp.float32),
            pltpu.SemaphoreType.DMA,
        ],
    )
    def _combine(yb_hbm, mi_hbm, kwx_hbm, sh_hbm, out_hbm,
                 l0_v, l1_v, kwx_v, r_v, acc_v, sem):
        wid = lax.axis_index("s") * NC + lax.axis_index("c")
        base = wid * TPW
        pltpu.sync_copy(mi_hbm.at[2, pl.ds(base, TPW)], l0_v)
        pltpu.sync_copy(mi_hbm.at[3, pl.ds(base, TPW)], l1_v)
        pltpu.sync_copy(kwx_hbm.at[pl.ds(base, TPW)], kwx_v)

        pltpu.async_copy(yb_hbm.at[l0_v], r_v, sem).wait()

        def scale0(i, carry):
            w = kwx_v[i, pl.ds(0, 16)]
            for cc in range(D_MODEL // 16):
                sl = pl.ds(cc * 16, 16)
                acc_v[i, sl] = w * r_v[i, sl]
            return carry

        lax.fori_loop(0, TPW, scale0, 0)

        pltpu.async_copy(yb_hbm.at[l1_v], r_v, sem).wait()

        def scale1(i, carry):
            w = kwx_v[i, pl.ds(16, 16)]
            for cc in range(D_MODEL // 16):
                sl = pl.ds(cc * 16, 16)
                acc_v[i, sl] = acc_v[i, sl] + w * r_v[i, sl]
            return carry

        lax.fori_loop(0, TPW, scale1, 0)

        pltpu.sync_copy(sh_hbm.at[pl.ds(base, TPW)], r_v)

        def addsh(i, carry):
            for cc in range(D_MODEL // 16):
                sl = pl.ds(cc * 16, 16)
                acc_v[i, sl] = acc_v[i, sl] + r_v[i, sl]
            return carry

        lax.fori_loop(0, TPW, addsh, 0)

        pltpu.sync_copy(acc_v, out_hbm.at[pl.ds(base, TPW)])

    return _combine


def _shared_body(x_ref, w1_ref, b1_ref, w2_ref, b2_ref, out_ref, acc_ref):
    j = pl.program_id(0)
    f32 = jnp.float32
    bf16 = jnp.bfloat16
    h = jax.nn.gelu(
        jnp.dot(x_ref[...].astype(bf16), w1_ref[...].astype(bf16),
                preferred_element_type=f32)
        + b1_ref[...])
    contrib = jnp.dot(h.astype(bf16), w2_ref[...].astype(bf16),
                      preferred_element_type=f32)

    @pl.when(j == 0)
    def _():
        acc_ref[...] = contrib + b2_ref[...]

    @pl.when(j > 0)
    def _():
        acc_ref[...] += contrib

    @pl.when(j == NFB - 1)
    def _():
        out_ref[...] = acc_ref[...]


def _shared(xf, Ws1, bs1, Ws2, bs2):
    return pl.pallas_call(
        _shared_body,
        grid=(NFB,),
        in_specs=[
            pl.BlockSpec((T, D_MODEL), lambda j: (0, 0)),
            pl.BlockSpec((D_MODEL, FB), lambda j: (0, j)),
            pl.BlockSpec((1, FB), lambda j: (0, j)),
            pl.BlockSpec((FB, D_MODEL), lambda j: (j, 0)),
            pl.BlockSpec((1, D_MODEL), lambda j: (0, 0)),
        ],
        out_specs=pl.BlockSpec((T, D_MODEL), lambda j: (0, 0)),
        out_shape=jax.ShapeDtypeStruct((T, D_MODEL), jnp.float32),
        scratch_shapes=[pltpu.VMEM((T, D_MODEL), jnp.float32)],
        compiler_params=pltpu.CompilerParams(
            dimension_semantics=("arbitrary",)),
    )(xf, Ws1, bs1, Ws2, bs2)


def kernel(x, Wg, bg, W1, b1, W2, b2, Ws1, bs1, Ws2, bs2):
    B, S, d = x.shape
    xf = x.reshape(T, d)

    wgp = jnp.zeros((D_MODEL, LANES), jnp.float32).at[:, :E].set(Wg)
    bgp = jnp.full((1, LANES), NEG, jnp.float32).at[0, :E].set(bg)

    mi, kwx = _router(xf, wgp, bgp)
    buf = _dispatch_kernel()(xf, mi)
    shared = _shared(xf, Ws1, bs1.reshape(1, D_FF), Ws2,
                     bs2.reshape(1, D_MODEL))
    yb = _experts(buf, W1, b1.reshape(E, 1, D_FF), W2,
                  b2.reshape(E, 1, D_MODEL))
    out = _combine_kernel()(yb, mi, kwx, shared)
    return out.reshape(B, S, d)
